# Initial kernel scaffold; baseline (speedup 1.0000x reference)
import jax
import jax.numpy as jnp
from jax.experimental import pallas as pl


def kernel(x, W, b):
    raise NotImplementedError("write your pallas kernel here")


# speedup vs baseline: 1.5798x; 1.5798x over previous
import functools

import jax
import jax.numpy as jnp
from jax import lax
from jax.experimental import pallas as pl
from jax.experimental.pallas import tpu as pltpu
from jax.experimental.pallas import tpu_sc as plsc

IN_F = 4096
OUT_F = 4096
TOPK = 1024
BSZ = 32

L = 16
NSUB = 16
FPW = IN_F // NSUB
NVR = FPW // L
N_ROUNDS = 8


NB = 256
HV = NB // L
NVALL = IN_F // L


def _select_body(x_hbm, mask_hbm, sc_hbm, xbuf, sbuf, bbuf, hist, mbuf):
    cid = lax.axis_index("c")
    sid = lax.axis_index("s")

    @pl.when(cid == 0)
    def _core0():
        pltpu.sync_copy(x_hbm.at[:, pl.ds(sid * FPW, FPW)], xbuf)

        def score_step(bi, accs):
            return tuple(
                accs[j] + jnp.abs(xbuf[bi, pl.ds(j * L, L)])
                for j in range(NVR)
            )

        accs = lax.fori_loop(
            0, BSZ, score_step,
            tuple(jnp.zeros((L,), jnp.float32) for _ in range(NVR)),
        )
        for j in range(NVR):
            sbuf[pl.ds(j * L, L)] = accs[j]
        pltpu.sync_copy(sbuf, sc_hbm.at[pl.ds(sid * FPW, FPW)])
        plsc.subcore_barrier()

        @pl.when(sid == 0)
        def _select():
            pltpu.sync_copy(sc_hbm, mbuf)
            for i in range(NVALL):
                bbuf[pl.ds(i * L, L)] = plsc.bitcast(
                    mbuf[pl.ds(i * L, L)], jnp.int32)

            iota = lax.iota(jnp.int32, L)
            ones = jnp.ones((L,), jnp.int32)
            zeros16 = jnp.zeros((L,), jnp.int32)

            prefix = jnp.int32(0)
            k_rem = jnp.int32(TOPK)
            for r in range(4):
                shift = 24 - 8 * r
                pmask = (jnp.int32(-(1 << (shift + 8))) if r > 0
                         else jnp.int32(0))
                for h in range(HV):
                    hist[pl.ds(h * L, L)] = zeros16

                def scan_step(i, c, pmask=pmask, prefix=prefix, shift=shift):
                    v = bbuf[pl.ds(i * L, L)]
                    match = (v & pmask) == prefix
                    bucket = lax.shift_right_logical(
                        v, jnp.int32(shift)) & (NB - 1)
                    plsc.addupdate_scatter(hist, [bucket], ones, mask=match)
                    return c

                lax.fori_loop(0, NVALL, scan_step, jnp.int32(0))

                jstar = jnp.int32(0)
                hsel = jnp.int32(0)
                cgsel = jnp.int32(0)
                found = jnp.zeros((), jnp.bool_)
                carry = jnp.int32(0)
                for h in range(HV - 1, -1, -1):
                    hv = hist[pl.ds(h * L, L)]
                    cg = lax.rev(plsc.cumsum(lax.rev(hv, (0,))), (0,)) + carry
                    ok = cg >= k_rem
                    p = plsc.all_reduce_population_count(ok)[0]
                    sel = iota == (p - 1)
                    hj = jnp.sum(jnp.where(sel, hv, 0))
                    cj = jnp.sum(jnp.where(sel, cg, 0))
                    hit = jnp.logical_and(jnp.logical_not(found), p > 0)
                    jstar = jnp.where(hit, h * L + p - 1, jstar)
                    hsel = jnp.where(hit, hj, hsel)
                    cgsel = jnp.where(hit, cj, cgsel)
                    found = jnp.logical_or(found, p > 0)
                    carry = carry + jnp.sum(hv)
                k_rem = k_rem - (cgsel - hsel)
                prefix = prefix | lax.shift_left(jstar, jnp.int32(shift))

            t_vec = jnp.zeros((L,), jnp.int32) + prefix

            def mask_step(i, carry):
                v = bbuf[pl.ds(i * L, L)]
                eq = v == t_vec
                eqi = jnp.where(eq, 1, 0)
                rank = plsc.cumsum(eqi) + carry
                keep = jnp.logical_and(eq, rank <= k_rem)
                gt = v > t_vec
                mbuf[pl.ds(i * L, L)] = jnp.where(
                    jnp.logical_or(gt, keep), 1.0, 0.0)
                return carry + jnp.sum(eqi)

            lax.fori_loop(0, NVALL, mask_step, jnp.int32(0))
            pltpu.sync_copy(mbuf, mask_hbm)


@functools.partial(
    pl.kernel,
    out_type=(
        jax.ShapeDtypeStruct((IN_F,), jnp.float32),
        jax.ShapeDtypeStruct((IN_F,), jnp.float32),
    ),
    mesh=plsc.VectorSubcoreMesh(
        core_axis_name="c", subcore_axis_name="s", num_cores=2,
        num_subcores=NSUB,
    ),
    scratch_types=[
        pltpu.VMEM((BSZ, FPW), jnp.float32),
        pltpu.VMEM((FPW,), jnp.float32),
        pltpu.VMEM((IN_F,), jnp.int32),
        pltpu.VMEM((NB,), jnp.int32),
        pltpu.VMEM((IN_F,), jnp.float32),
    ],
    compiler_params=pltpu.CompilerParams(needs_layout_passes=False),
)
def _topk_mask_sc(x_hbm, mask_hbm, sc_hbm, xbuf, sbuf, bbuf, hist, mbuf):
    _select_body(x_hbm, mask_hbm, sc_hbm, xbuf, sbuf, bbuf, hist, mbuf)


BN = 512


def _mm_body(x_ref, m_ref, w_ref, b_ref, o_ref):
    xm = x_ref[...] * m_ref[...]
    acc = lax.dot_general(
        xm, w_ref[...], (((1,), (1,)), ((), ())),
        preferred_element_type=jnp.float32,
    )
    o_ref[...] = acc + b_ref[...]


def _masked_matmul(x2, mask, W, b):
    return pl.pallas_call(
        _mm_body,
        grid=(OUT_F // BN,),
        in_specs=[
            pl.BlockSpec((BSZ, IN_F), lambda i: (0, 0)),
            pl.BlockSpec((1, IN_F), lambda i: (0, 0)),
            pl.BlockSpec((BN, IN_F), lambda i: (i, 0)),
            pl.BlockSpec((1, BN), lambda i: (0, i)),
        ],
        out_specs=pl.BlockSpec((BSZ, BN), lambda i: (0, i)),
        out_shape=jax.ShapeDtypeStruct((BSZ, OUT_F), jnp.float32),
    )(x2, mask.reshape(1, IN_F), W, b.reshape(1, OUT_F))


def kernel(x, W, b):
    x2 = x.reshape(BSZ, IN_F)
    mask, _ = _topk_mask_sc(x2)
    out = _masked_matmul(x2, mask, W, b)
    return out.reshape(BSZ, 1, OUT_F)
